# Initial kernel scaffold; baseline (speedup 1.0000x reference)
import jax
import jax.numpy as jnp
from jax.experimental import pallas as pl


def kernel(input_ids, emb, W1, b1, W2, b2):
    raise NotImplementedError("write your pallas kernel here")


# speedup vs baseline: 7.5001x; 7.5001x over previous
import functools

import jax
import jax.numpy as jnp
from jax import lax
from jax.experimental import pallas as pl
from jax.experimental.pallas import tpu as pltpu
from jax.experimental.pallas import tpu_sc as plsc

VOCAB = 100000
EMB_DIM = 128
HIDDEN = 256
NUM_CLASSES = 100
B = 4096
L = 200

NUM_CORES = 2
NUM_SUBCORES = 16
NW = NUM_CORES * NUM_SUBCORES
BPW = B // NW
G1 = 104
G2 = L - G1
LANES = 16
NV = EMB_DIM // LANES


def _pool_body(ids_hbm, emb_hbm, out_hbm, idx_v, rows_v, out_v, sem):
    c = lax.axis_index("c")
    s = lax.axis_index("s")
    wid = s * NUM_CORES + c
    base = pl.multiple_of(wid * BPW, 8)
    pltpu.sync_copy(
        ids_hbm.at[pl.ds(pl.multiple_of(wid * (BPW * L), 8), BPW * L)], idx_v)

    def row_body(b, _):
        off1 = pl.multiple_of(b * L, 8)
        off2 = pl.multiple_of(b * L + G1, 8)
        cp1 = pltpu.async_copy(
            emb_hbm.at[idx_v.at[pl.ds(off1, G1)]], rows_v.at[pl.ds(0, G1)], sem)
        cp2 = pltpu.async_copy(
            emb_hbm.at[idx_v.at[pl.ds(off2, G2)]], rows_v.at[pl.ds(G1, G2)], sem)
        cp1.wait()
        cp2.wait()

        def acc_body(j, acc):
            return tuple(acc[k] + rows_v[j, pl.ds(k * LANES, LANES)]
                         for k in range(NV))

        acc = lax.fori_loop(
            0, L, acc_body,
            tuple(jnp.zeros((LANES,), jnp.float32) for _ in range(NV)))
        for k in range(NV):
            out_v[b, pl.ds(k * LANES, LANES)] = acc[k]
        return _

    lax.fori_loop(0, BPW, row_body, 0)
    pltpu.sync_copy(out_v, out_hbm.at[pl.ds(base, BPW)])


@functools.partial(jax.jit, static_argnames=())
def _pool(ids, emb):
    mesh = plsc.VectorSubcoreMesh(core_axis_name="c", subcore_axis_name="s")
    return pl.kernel(
        _pool_body,
        out_type=jax.ShapeDtypeStruct((B, EMB_DIM), jnp.float32),
        mesh=mesh,
        scratch_types=[
            pltpu.VMEM((BPW * L,), jnp.int32),
            pltpu.VMEM((L, EMB_DIM), jnp.float32),
            pltpu.VMEM((BPW, EMB_DIM), jnp.float32),
            pltpu.SemaphoreType.DMA,
        ],
    )(ids, emb)


def _mlp_body(x_ref, w1_ref, b1_ref, w2_ref, b2_ref, o_ref):
    x = x_ref[...] * (1.0 / L)
    h = jnp.dot(x, w1_ref[...], preferred_element_type=jnp.float32)
    h = jnp.maximum(h + b1_ref[...], 0.0)
    o = jnp.dot(h, w2_ref[...], preferred_element_type=jnp.float32)
    o_ref[...] = o + b2_ref[...]


def _mlp(x, w1, b1, w2, b2):
    bt = 1024
    return pl.pallas_call(
        _mlp_body,
        grid=(B // bt,),
        in_specs=[
            pl.BlockSpec((bt, EMB_DIM), lambda i: (i, 0)),
            pl.BlockSpec((EMB_DIM, HIDDEN), lambda i: (0, 0)),
            pl.BlockSpec((1, HIDDEN), lambda i: (0, 0)),
            pl.BlockSpec((HIDDEN, NUM_CLASSES), lambda i: (0, 0)),
            pl.BlockSpec((1, NUM_CLASSES), lambda i: (0, 0)),
        ],
        out_specs=pl.BlockSpec((bt, NUM_CLASSES), lambda i: (i, 0)),
        out_shape=jax.ShapeDtypeStruct((B, NUM_CLASSES), jnp.float32),
    )(x, w1, b1, w2, b2)


def kernel(input_ids, emb, W1, b1, W2, b2):
    ids = input_ids.astype(jnp.int32).reshape(B * L)
    pooled = _pool(ids, emb)
    return _mlp(pooled, W1, b1.reshape(1, HIDDEN), W2, b2.reshape(1, NUM_CLASSES))


# speedup vs baseline: 12.9429x; 1.7257x over previous
import functools

import jax
import jax.numpy as jnp
from jax import lax
from jax.experimental import pallas as pl
from jax.experimental.pallas import tpu as pltpu
from jax.experimental.pallas import tpu_sc as plsc

VOCAB = 100000
EMB_DIM = 128
HIDDEN = 256
NUM_CLASSES = 100
B = 4096
L = 200

NUM_CORES = 2
NUM_SUBCORES = 16
NW = NUM_CORES * NUM_SUBCORES
BPW = B // NW
G1 = 104
G2 = L - G1
LANES = 16
NV = EMB_DIM // LANES


UNROLL = 4


def _pool_body(ids_hbm, emb_hbm, out_hbm, idx_v, rows_v, out_v, sem0, sem1):
    c = lax.axis_index("c")
    s = lax.axis_index("s")
    wid = s * NUM_CORES + c
    base = pl.multiple_of(wid * BPW, 8)
    pltpu.sync_copy(
        ids_hbm.at[pl.ds(pl.multiple_of(wid * (BPW * L), 8), BPW * L)], idx_v)

    sems = (sem0, sem1)

    def issue(b, buf):
        off1 = pl.multiple_of(b * L, 8)
        off2 = pl.multiple_of(b * L + G1, 8)
        pltpu.async_copy(
            emb_hbm.at[idx_v.at[pl.ds(off1, G1)]],
            rows_v.at[buf, pl.ds(0, G1)], sems[buf])
        pltpu.async_copy(
            emb_hbm.at[idx_v.at[pl.ds(off2, G2)]],
            rows_v.at[buf, pl.ds(G1, G2)], sems[buf])

    def drain(buf):
        pltpu.make_async_copy(
            emb_hbm.at[pl.ds(0, L)], rows_v.at[buf], sems[buf]).wait()

    def accum(b, buf):
        def acc_body(jj, acc):
            for r in range(UNROLL):
                j = jj * UNROLL + r
                acc = tuple(acc[k] + rows_v[buf, j, pl.ds(k * LANES, LANES)]
                            for k in range(NV))
            return acc

        acc = lax.fori_loop(
            0, L // UNROLL, acc_body,
            tuple(jnp.zeros((LANES,), jnp.float32) for _ in range(NV)))
        for k in range(NV):
            out_v[b, pl.ds(k * LANES, LANES)] = acc[k]

    issue(0, 0)

    def pair_body(bb, _):
        b0 = bb * 2
        issue(b0 + 1, 1)
        drain(0)
        accum(b0, 0)

        @pl.when(bb + 1 < BPW // 2)
        def _issue_next():
            issue(b0 + 2, 0)

        drain(1)
        accum(b0 + 1, 1)
        return _

    lax.fori_loop(0, BPW // 2, pair_body, 0)
    pltpu.sync_copy(out_v, out_hbm.at[pl.ds(base, BPW)])


@functools.partial(jax.jit, static_argnames=())
def _pool(ids, emb):
    mesh = plsc.VectorSubcoreMesh(core_axis_name="c", subcore_axis_name="s")
    return pl.kernel(
        _pool_body,
        out_type=jax.ShapeDtypeStruct((B, EMB_DIM), jnp.float32),
        mesh=mesh,
        scratch_types=[
            pltpu.VMEM((BPW * L,), jnp.int32),
            pltpu.VMEM((2, L, EMB_DIM), jnp.float32),
            pltpu.VMEM((BPW, EMB_DIM), jnp.float32),
            pltpu.SemaphoreType.DMA,
            pltpu.SemaphoreType.DMA,
        ],
    )(ids, emb)


def _mlp_body(x_ref, w1_ref, b1_ref, w2_ref, b2_ref, o_ref):
    x = x_ref[...] * (1.0 / L)
    h = jnp.dot(x, w1_ref[...], preferred_element_type=jnp.float32)
    h = jnp.maximum(h + b1_ref[...], 0.0)
    o = jnp.dot(h, w2_ref[...], preferred_element_type=jnp.float32)
    o_ref[...] = o + b2_ref[...]


def _mlp(x, w1, b1, w2, b2):
    bt = 1024
    return pl.pallas_call(
        _mlp_body,
        grid=(B // bt,),
        in_specs=[
            pl.BlockSpec((bt, EMB_DIM), lambda i: (i, 0)),
            pl.BlockSpec((EMB_DIM, HIDDEN), lambda i: (0, 0)),
            pl.BlockSpec((1, HIDDEN), lambda i: (0, 0)),
            pl.BlockSpec((HIDDEN, NUM_CLASSES), lambda i: (0, 0)),
            pl.BlockSpec((1, NUM_CLASSES), lambda i: (0, 0)),
        ],
        out_specs=pl.BlockSpec((bt, NUM_CLASSES), lambda i: (i, 0)),
        out_shape=jax.ShapeDtypeStruct((B, NUM_CLASSES), jnp.float32),
    )(x, w1, b1, w2, b2)


def kernel(input_ids, emb, W1, b1, W2, b2):
    ids = input_ids.astype(jnp.int32).reshape(B * L)
    pooled = _pool(ids, emb)
    return _mlp(pooled, W1, b1.reshape(1, HIDDEN), W2, b2.reshape(1, NUM_CLASSES))


# speedup vs baseline: 15.6252x; 1.2072x over previous
import functools

import jax
import jax.numpy as jnp
from jax import lax
from jax.experimental import pallas as pl
from jax.experimental.pallas import tpu as pltpu
from jax.experimental.pallas import tpu_sc as plsc

VOCAB = 100000
EMB_DIM = 128
HIDDEN = 256
NUM_CLASSES = 100
B = 4096
L = 200

NUM_CORES = 2
NUM_SUBCORES = 16
NW = NUM_CORES * NUM_SUBCORES
BPW = B // NW
G1 = 104
G2 = L - G1
LANES = 16
NV = EMB_DIM // LANES


UNROLL = 8


def _pool_body(ids_hbm, emb_hbm, out_hbm, idx_v, rows_v, out_v,
               sem0, sem1, sem2, sem3):
    c = lax.axis_index("c")
    s = lax.axis_index("s")
    wid = s * NUM_CORES + c
    base = pl.multiple_of(wid * BPW, 8)
    pltpu.sync_copy(
        ids_hbm.at[pl.ds(pl.multiple_of(wid * (BPW * L), 8), BPW * L)], idx_v)

    sems = (sem0, sem1, sem2, sem3)
    sizes = (G1, G2)

    def issue(b, half, buf):
        off = pl.multiple_of(b * L + half * G1, 8)
        pltpu.async_copy(
            emb_hbm.at[idx_v.at[pl.ds(off, sizes[half])]],
            rows_v.at[buf, pl.ds(0, sizes[half])], sems[buf])

    def drain(half, buf):
        pltpu.make_async_copy(
            emb_hbm.at[pl.ds(0, sizes[half])],
            rows_v.at[buf, pl.ds(0, sizes[half])], sems[buf]).wait()

    def accum(half, buf, acc):
        def acc_body(jj, a):
            for r in range(UNROLL):
                j = jj * UNROLL + r
                a = tuple(a[k] + rows_v[buf, j, pl.ds(k * LANES, LANES)]
                          for k in range(NV))
            return a

        return lax.fori_loop(0, sizes[half] // UNROLL, acc_body, acc)

    issue(0, 0, 0)
    issue(0, 1, 1)
    issue(1, 0, 2)
    issue(1, 1, 3)

    def pair_body(bb, _):
        zeros = tuple(jnp.zeros((LANES,), jnp.float32) for _ in range(NV))
        for pr in range(2):
            b = bb * 2 + pr
            buf0 = 2 * pr
            buf1 = 2 * pr + 1

            drain(0, buf0)
            acc = accum(0, buf0, zeros)

            @pl.when(b + 2 < BPW)
            def _issue_a():
                issue(b + 2, 0, buf0)

            drain(1, buf1)
            acc = accum(1, buf1, acc)

            @pl.when(b + 2 < BPW)
            def _issue_b():
                issue(b + 2, 1, buf1)

            for k in range(NV):
                out_v[b, pl.ds(k * LANES, LANES)] = acc[k]
        return _

    lax.fori_loop(0, BPW // 2, pair_body, 0)
    pltpu.sync_copy(out_v, out_hbm.at[pl.ds(base, BPW)])


@functools.partial(jax.jit, static_argnames=())
def _pool(ids, emb):
    mesh = plsc.VectorSubcoreMesh(core_axis_name="c", subcore_axis_name="s")
    return pl.kernel(
        _pool_body,
        out_type=jax.ShapeDtypeStruct((B, EMB_DIM), jnp.float32),
        mesh=mesh,
        scratch_types=[
            pltpu.VMEM((BPW * L,), jnp.int32),
            pltpu.VMEM((4, G1, EMB_DIM), jnp.float32),
            pltpu.VMEM((BPW, EMB_DIM), jnp.float32),
            pltpu.SemaphoreType.DMA,
            pltpu.SemaphoreType.DMA,
            pltpu.SemaphoreType.DMA,
            pltpu.SemaphoreType.DMA,
        ],
    )(ids, emb)


def _mlp_body(x_ref, w1_ref, b1_ref, w2_ref, b2_ref, o_ref):
    x = x_ref[...] * (1.0 / L)
    h = jnp.dot(x, w1_ref[...], preferred_element_type=jnp.float32)
    h = jnp.maximum(h + b1_ref[...], 0.0)
    o = jnp.dot(h, w2_ref[...], preferred_element_type=jnp.float32)
    o_ref[...] = o + b2_ref[...]


def _mlp(x, w1, b1, w2, b2):
    bt = 1024
    return pl.pallas_call(
        _mlp_body,
        grid=(B // bt,),
        in_specs=[
            pl.BlockSpec((bt, EMB_DIM), lambda i: (i, 0)),
            pl.BlockSpec((EMB_DIM, HIDDEN), lambda i: (0, 0)),
            pl.BlockSpec((1, HIDDEN), lambda i: (0, 0)),
            pl.BlockSpec((HIDDEN, NUM_CLASSES), lambda i: (0, 0)),
            pl.BlockSpec((1, NUM_CLASSES), lambda i: (0, 0)),
        ],
        out_specs=pl.BlockSpec((bt, NUM_CLASSES), lambda i: (i, 0)),
        out_shape=jax.ShapeDtypeStruct((B, NUM_CLASSES), jnp.float32),
    )(x, w1, b1, w2, b2)


def kernel(input_ids, emb, W1, b1, W2, b2):
    ids = input_ids.astype(jnp.int32).reshape(B * L)
    pooled = _pool(ids, emb)
    return _mlp(pooled, W1, b1.reshape(1, HIDDEN), W2, b2.reshape(1, NUM_CLASSES))
